# baseline (speedup 1.0000x reference)
import jax
import jax.numpy as jnp
from jax import lax
from jax.experimental import pallas as pl
from jax.experimental.pallas import tpu as pltpu
from jax.experimental.pallas import tpu_sc as plsc

N_NODES = 100000
N_LINKS = 6400000
NW = 32
LINKS_PER_W = N_LINKS // NW
C = 800
NCHUNK = LINKS_PER_W // C
G = NCHUNK // 2
U = 5
NP = 100352
RSL = NP // 16
RSL8 = RSL // 8


def _i32(x):
    return jnp.int32(x)


def _sc_body(ei, vel, flen, ctrl, field,
             partial, sel_hbm, ff_hbm, accs,
             big, ti0, ti1, hi0, hi1, si0, si1,
             vf0, vf1, wf0, wf1, ff0, ff1,
             isem0, isem1, osem0, osem1):
    cid = lax.axis_index("c").astype(jnp.int32)
    sid = lax.axis_index("s").astype(jnp.int32)
    wid = cid * _i32(16) + sid
    lbase = wid * _i32(LINKS_PER_W)

    tis = (ti0, ti1)
    his = (hi0, hi1)
    sis = (si0, si1)
    vfs = (vf0, vf1)
    wfs = (wf0, wf1)
    ffs = (ff0, ff1)
    isems = (isem0, isem1)
    osems = (osem0, osem1)

    def run_phase(start_in, wait_in, compute, start_out, wait_out):
        start_in(_i32(0), 0)

        def body(g, carry):
            c0 = g * _i32(2)
            c1 = c0 + _i32(1)
            start_in(c1, 1)
            wait_in(c0, 0)
            if wait_out is not None:
                @pl.when(g > _i32(0))
                def _():
                    wait_out(c0 - _i32(2), 0)
            compute(0)
            if start_out is not None:
                start_out(c0, 0)

            @pl.when(g < _i32(G - 1))
            def _():
                start_in(c0 + _i32(2), 0)

            wait_in(c1, 1)
            if wait_out is not None:
                @pl.when(g > _i32(0))
                def _():
                    wait_out(c1 - _i32(2), 1)
            compute(1)
            if start_out is not None:
                start_out(c1, 1)
            return carry

        lax.fori_loop(_i32(0), _i32(G), body, _i32(0))
        if wait_out is not None:
            wait_out(_i32(NCHUNK - 2), 0)
            wait_out(_i32(NCHUNK - 1), 1)

    pltpu.sync_copy(ctrl, big.at[pl.ds(0, N_NODES)])

    i2 = jax.lax.iota(jnp.int32, 16) * _i32(2)

    def p1_start_in(ci, s):
        cb2 = (lbase + ci * _i32(C)) * _i32(2)
        pltpu.async_copy(ei.at[pl.ds(cb2, 2 * C)], tis[s], isems[s])
        pltpu.async_copy(ei.at[pl.ds(_i32(2 * N_LINKS) + cb2, 2 * C)],
                         his[s], isems[s])

    def p1_wait_in(ci, s):
        cb2 = (lbase + ci * _i32(C)) * _i32(2)
        pltpu.make_async_copy(ei.at[pl.ds(cb2, 2 * C)], tis[s],
                              isems[s]).wait()
        pltpu.make_async_copy(ei.at[pl.ds(_i32(2 * N_LINKS) + cb2, 2 * C)],
                              his[s], isems[s]).wait()

    def p1_compute(s):
        tb, hb, sb = tis[s], his[s], sis[s]

        def inner(j, c2):
            ob = j * _i32(16 * U)
            for k in range(U):
                o = ob + _i32(16 * k)
                idx2 = i2 + o * _i32(2)
                t = plsc.load_gather(tb, [idx2])
                h = plsc.load_gather(hb, [idx2])
                ct = plsc.load_gather(big, [t])
                ch = plsc.load_gather(big, [h])
                sb[pl.ds(o, 16)] = jnp.where(ch > ct, h, t)
            return c2

        lax.fori_loop(_i32(0), _i32(C // (16 * U)), inner, _i32(0))

    def p1_start_out(ci, s):
        cb = lbase + ci * _i32(C)
        pltpu.async_copy(sis[s], sel_hbm.at[pl.ds(cb, C)], osems[s])

    def p1_wait_out(ci, s):
        cb = lbase + ci * _i32(C)
        pltpu.make_async_copy(sis[s], sel_hbm.at[pl.ds(cb, C)],
                              osems[s]).wait()

    run_phase(p1_start_in, p1_wait_in, p1_compute, p1_start_out, p1_wait_out)

    pltpu.sync_copy(field, big.at[pl.ds(0, N_NODES)])

    def p2_start_in(ci, s):
        cb = lbase + ci * _i32(C)
        pltpu.async_copy(sel_hbm.at[pl.ds(cb, C)], sis[s], isems[s])
        pltpu.async_copy(vel.at[pl.ds(cb, C)], vfs[s], isems[s])
        pltpu.async_copy(flen.at[pl.ds(cb, C)], wfs[s], isems[s])

    def p2_wait_in(ci, s):
        cb = lbase + ci * _i32(C)
        pltpu.make_async_copy(sel_hbm.at[pl.ds(cb, C)], sis[s],
                              isems[s]).wait()
        pltpu.make_async_copy(vel.at[pl.ds(cb, C)], vfs[s], isems[s]).wait()
        pltpu.make_async_copy(flen.at[pl.ds(cb, C)], wfs[s], isems[s]).wait()

    def p2_compute(s):
        sb, vb, wb, fb = sis[s], vfs[s], wfs[s], ffs[s]

        def inner(j, c2):
            ob = j * _i32(16 * U)
            for k in range(U):
                o = ob + _i32(16 * k)
                sidx = sb[pl.ds(o, 16)]
                fv = plsc.load_gather(big, [sidx])
                fb[pl.ds(o, 16)] = fv * vb[pl.ds(o, 16)] * wb[pl.ds(o, 16)]
            return c2

        lax.fori_loop(_i32(0), _i32(C // (16 * U)), inner, _i32(0))

    def p2_start_out(ci, s):
        cb = lbase + ci * _i32(C)
        pltpu.async_copy(ffs[s], ff_hbm.at[pl.ds(cb, C)], osems[s])

    def p2_wait_out(ci, s):
        cb = lbase + ci * _i32(C)
        pltpu.make_async_copy(ffs[s], ff_hbm.at[pl.ds(cb, C)],
                              osems[s]).wait()

    run_phase(p2_start_in, p2_wait_in, p2_compute, p2_start_out, p2_wait_out)

    def zero(j, carry):
        ob = j * _i32(128)
        for k in range(8):
            big[pl.ds(ob + _i32(16 * k), 16)] = jnp.zeros((16,), jnp.float32)
        return carry

    lax.fori_loop(_i32(0), _i32(NP // 128), zero, _i32(0))

    def p3_start_in(ci, s):
        cb = lbase + ci * _i32(C)
        cb2 = cb * _i32(2)
        pltpu.async_copy(ei.at[pl.ds(cb2, 2 * C)], tis[s], isems[s])
        pltpu.async_copy(ei.at[pl.ds(_i32(2 * N_LINKS) + cb2, 2 * C)],
                         his[s], isems[s])
        pltpu.async_copy(ff_hbm.at[pl.ds(cb, C)], ffs[s], isems[s])

    def p3_wait_in(ci, s):
        cb = lbase + ci * _i32(C)
        cb2 = cb * _i32(2)
        pltpu.make_async_copy(ei.at[pl.ds(cb2, 2 * C)], tis[s],
                              isems[s]).wait()
        pltpu.make_async_copy(ei.at[pl.ds(_i32(2 * N_LINKS) + cb2, 2 * C)],
                              his[s], isems[s]).wait()
        pltpu.make_async_copy(ff_hbm.at[pl.ds(cb, C)], ffs[s],
                              isems[s]).wait()

    def p3_compute(s):
        tb, hb, fb = tis[s], his[s], ffs[s]

        def inner(j, c2):
            ob = j * _i32(16 * U)
            for k in range(U):
                o = ob + _i32(16 * k)
                idx2 = i2 + o * _i32(2)
                t = plsc.load_gather(tb, [idx2])
                h = plsc.load_gather(hb, [idx2])
                ffv = fb[pl.ds(o, 16)]
                plsc.addupdate_scatter(big, [t], ffv)
                plsc.addupdate_scatter(big, [h], -ffv)
            return c2

        lax.fori_loop(_i32(0), _i32(C // (16 * U)), inner, _i32(0))

    run_phase(p3_start_in, p3_wait_in, p3_compute, None, None)

    pltpu.sync_copy(big, accs.at[pl.ds(wid * _i32(NP), NP)])
    plsc.subcore_barrier()
    for r in range(8):
        rb = sid * _i32(RSL) + _i32(r * RSL8)
        pltpu.sync_copy(accs.at[pl.ds(cid * _i32(16 * NP) + rb, RSL8)],
                        vf0.at[pl.ds(0, RSL8)])

        def red_one(k, carry):
            off = (cid * _i32(16) + k) * _i32(NP) + rb
            pltpu.sync_copy(accs.at[pl.ds(off, RSL8)], wf0.at[pl.ds(0, RSL8)])

            def add16(j, c2):
                ob = j * _i32(112)
                for k2 in range(7):
                    o = ob + _i32(16 * k2)
                    vf0[pl.ds(o, 16)] = vf0[pl.ds(o, 16)] + wf0[pl.ds(o, 16)]
                return c2

            lax.fori_loop(_i32(0), _i32(RSL8 // 112), add16, _i32(0))
            return carry

        lax.fori_loop(_i32(1), _i32(16), red_one, _i32(0))
        pltpu.sync_copy(vf0.at[pl.ds(0, RSL8)],
                        partial.at[pl.ds(cid * _i32(NP) + rb, RSL8)])


@jax.jit
def _sc_part(ei, vel, flen, ctrl, field):
    mesh = plsc.VectorSubcoreMesh(core_axis_name="c", subcore_axis_name="s")
    f = pl.kernel(
        _sc_body,
        out_type=[
            jax.ShapeDtypeStruct((2 * NP,), jnp.float32),
            jax.ShapeDtypeStruct((N_LINKS,), jnp.int32),
            jax.ShapeDtypeStruct((N_LINKS,), jnp.float32),
            jax.ShapeDtypeStruct((NW * NP,), jnp.float32),
        ],
        mesh=mesh,
        compiler_params=pltpu.CompilerParams(needs_layout_passes=False),
        scratch_types=[
            pltpu.VMEM((NP,), jnp.float32),
            pltpu.VMEM((2 * C,), jnp.int32),
            pltpu.VMEM((2 * C,), jnp.int32),
            pltpu.VMEM((2 * C,), jnp.int32),
            pltpu.VMEM((2 * C,), jnp.int32),
            pltpu.VMEM((C,), jnp.int32),
            pltpu.VMEM((C,), jnp.int32),
            pltpu.VMEM((C,), jnp.float32),
            pltpu.VMEM((C,), jnp.float32),
            pltpu.VMEM((C,), jnp.float32),
            pltpu.VMEM((C,), jnp.float32),
            pltpu.VMEM((C,), jnp.float32),
            pltpu.VMEM((C,), jnp.float32),
            pltpu.SemaphoreType.DMA,
            pltpu.SemaphoreType.DMA,
            pltpu.SemaphoreType.DMA,
            pltpu.SemaphoreType.DMA,
        ],
    )
    return f(ei, vel, flen, ctrl, field)


def _tc_combine_body(f_ref, a_ref, p0_ref, p1_ref, dt_ref, o_ref):
    dt = dt_ref[0]
    o_ref[...] = f_ref[...] - dt * (p0_ref[...] + p1_ref[...]) / a_ref[...]


@jax.jit
def _tc_combine(fp, ap, p0, p1, dt):
    return pl.pallas_call(
        _tc_combine_body,
        out_shape=jax.ShapeDtypeStruct((NP // 1024, 1024), jnp.float32),
        in_specs=[
            pl.BlockSpec(memory_space=pltpu.MemorySpace.VMEM),
            pl.BlockSpec(memory_space=pltpu.MemorySpace.VMEM),
            pl.BlockSpec(memory_space=pltpu.MemorySpace.VMEM),
            pl.BlockSpec(memory_space=pltpu.MemorySpace.VMEM),
            pl.BlockSpec(memory_space=pltpu.MemorySpace.SMEM),
        ],
        out_specs=pl.BlockSpec(memory_space=pltpu.MemorySpace.VMEM),
    )(fp, ap, p0, p1, dt)


def kernel(field, control, velocity, edge_index, length_of_face,
           cell_area_at_node, dt):
    ei = jnp.reshape(jax.lax.bitcast_convert_type(edge_index, jnp.int32),
                     (4 * N_LINKS,))
    partial, _sel, _ff, _accs = _sc_part(ei, velocity,
                                         length_of_face, control, field)
    pad = NP - N_NODES
    fp = jnp.reshape(jnp.pad(field, (0, pad)), (NP // 1024, 1024))
    ap = jnp.reshape(jnp.pad(cell_area_at_node, (0, pad),
                             constant_values=jnp.float32(1.0)),
                     (NP // 1024, 1024))
    p0 = jnp.reshape(partial[:NP], (NP // 1024, 1024))
    p1 = jnp.reshape(partial[NP:], (NP // 1024, 1024))
    dt_arr = jnp.reshape(dt.astype(jnp.float32), (1,))
    out = _tc_combine(fp, ap, p0, p1, dt_arr)
    return jnp.reshape(out, (NP,))[:N_NODES]


# speedup vs baseline: 21.4754x; 21.4754x over previous
import jax
import jax.numpy as jnp
from jax import lax
from jax.experimental import pallas as pl
from jax.experimental.pallas import tpu as pltpu
from jax.experimental.pallas import tpu_sc as plsc

N_NODES = 100000
N_LINKS = 6400000
NW = 32
LINKS_PER_W = N_LINKS // NW
C = 2000
NCHUNK = LINKS_PER_W // C
G = NCHUNK // 2
U = 5
NP = 100352
RSL = NP // 16
RSL4 = RSL // 4


def _i32(x):
    return jnp.int32(x)


def _sc_body(tail, head, vel, flen, ctrl, field,
             partial,
             sel_hbm, ff_hbm, accs,
             big, ti0, ti1, hi0, hi1, si0, si1,
             vf0, vf1, wf0, wf1, ff0, ff1,
             isem0, isem1, osem0, osem1):
    cid = lax.axis_index("c").astype(jnp.int32)
    sid = lax.axis_index("s").astype(jnp.int32)
    wid = cid * _i32(16) + sid
    lbase = wid * _i32(LINKS_PER_W)

    tis = (ti0, ti1)
    his = (hi0, hi1)
    sis = (si0, si1)
    vfs = (vf0, vf1)
    wfs = (wf0, wf1)
    ffs = (ff0, ff1)
    isems = (isem0, isem1)
    osems = (osem0, osem1)

    def run_phase(start_in, wait_in, compute, start_out, wait_out):
        start_in(_i32(0), 0)

        def body(g, carry):
            c0 = g * _i32(2)
            c1 = c0 + _i32(1)
            start_in(c1, 1)
            wait_in(c0, 0)
            if wait_out is not None:
                @pl.when(g > _i32(0))
                def _():
                    wait_out(c0 - _i32(2), 0)
            compute(0)
            if start_out is not None:
                start_out(c0, 0)

            @pl.when(g < _i32(G - 1))
            def _():
                start_in(c0 + _i32(2), 0)

            wait_in(c1, 1)
            if wait_out is not None:
                @pl.when(g > _i32(0))
                def _():
                    wait_out(c1 - _i32(2), 1)
            compute(1)
            if start_out is not None:
                start_out(c1, 1)
            return carry

        lax.fori_loop(_i32(0), _i32(G), body, _i32(0))
        if wait_out is not None:
            wait_out(_i32(NCHUNK - 2), 0)
            wait_out(_i32(NCHUNK - 1), 1)

    pltpu.sync_copy(ctrl, big.at[pl.ds(0, N_NODES)])

    def p1_start_in(ci, s):
        cb = lbase + ci * _i32(C)
        pltpu.async_copy(tail.at[pl.ds(cb, C)], tis[s], isems[s])
        pltpu.async_copy(head.at[pl.ds(cb, C)], his[s], isems[s])

    def p1_wait_in(ci, s):
        cb = lbase + ci * _i32(C)
        pltpu.make_async_copy(tail.at[pl.ds(cb, C)], tis[s], isems[s]).wait()
        pltpu.make_async_copy(head.at[pl.ds(cb, C)], his[s], isems[s]).wait()

    def p1_compute(s):
        tb, hb, sb = tis[s], his[s], sis[s]

        def inner(j, c2):
            ob = j * _i32(16 * U)
            for k in range(U):
                o = ob + _i32(16 * k)
                t = tb[pl.ds(o, 16)]
                h = hb[pl.ds(o, 16)]
                ct = plsc.load_gather(big, [t])
                ch = plsc.load_gather(big, [h])
                sb[pl.ds(o, 16)] = jnp.where(ch > ct, h, t)
            return c2

        lax.fori_loop(_i32(0), _i32(C // (16 * U)), inner, _i32(0))

    def p1_start_out(ci, s):
        cb = lbase + ci * _i32(C)
        pltpu.async_copy(sis[s], sel_hbm.at[pl.ds(cb, C)], osems[s])

    def p1_wait_out(ci, s):
        cb = lbase + ci * _i32(C)
        pltpu.make_async_copy(sis[s], sel_hbm.at[pl.ds(cb, C)],
                              osems[s]).wait()

    run_phase(p1_start_in, p1_wait_in, p1_compute, p1_start_out, p1_wait_out)

    pltpu.sync_copy(field, big.at[pl.ds(0, N_NODES)])

    def p2_start_in(ci, s):
        cb = lbase + ci * _i32(C)
        pltpu.async_copy(sel_hbm.at[pl.ds(cb, C)], sis[s], isems[s])
        pltpu.async_copy(vel.at[pl.ds(cb, C)], vfs[s], isems[s])
        pltpu.async_copy(flen.at[pl.ds(cb, C)], wfs[s], isems[s])

    def p2_wait_in(ci, s):
        cb = lbase + ci * _i32(C)
        pltpu.make_async_copy(sel_hbm.at[pl.ds(cb, C)], sis[s],
                              isems[s]).wait()
        pltpu.make_async_copy(vel.at[pl.ds(cb, C)], vfs[s], isems[s]).wait()
        pltpu.make_async_copy(flen.at[pl.ds(cb, C)], wfs[s], isems[s]).wait()

    def p2_compute(s):
        sb, vb, wb, fb = sis[s], vfs[s], wfs[s], ffs[s]

        def inner(j, c2):
            ob = j * _i32(16 * U)
            for k in range(U):
                o = ob + _i32(16 * k)
                sidx = sb[pl.ds(o, 16)]
                fv = plsc.load_gather(big, [sidx])
                fb[pl.ds(o, 16)] = fv * vb[pl.ds(o, 16)] * wb[pl.ds(o, 16)]
            return c2

        lax.fori_loop(_i32(0), _i32(C // (16 * U)), inner, _i32(0))

    def p2_start_out(ci, s):
        cb = lbase + ci * _i32(C)
        pltpu.async_copy(ffs[s], ff_hbm.at[pl.ds(cb, C)], osems[s])

    def p2_wait_out(ci, s):
        cb = lbase + ci * _i32(C)
        pltpu.make_async_copy(ffs[s], ff_hbm.at[pl.ds(cb, C)],
                              osems[s]).wait()

    run_phase(p2_start_in, p2_wait_in, p2_compute, p2_start_out, p2_wait_out)

    def zero(j, carry):
        ob = j * _i32(128)
        for k in range(8):
            big[pl.ds(ob + _i32(16 * k), 16)] = jnp.zeros((16,), jnp.float32)
        return carry

    lax.fori_loop(_i32(0), _i32(NP // 128), zero, _i32(0))

    def p3_start_in(ci, s):
        cb = lbase + ci * _i32(C)
        pltpu.async_copy(tail.at[pl.ds(cb, C)], tis[s], isems[s])
        pltpu.async_copy(head.at[pl.ds(cb, C)], his[s], isems[s])
        pltpu.async_copy(ff_hbm.at[pl.ds(cb, C)], ffs[s], isems[s])

    def p3_wait_in(ci, s):
        cb = lbase + ci * _i32(C)
        pltpu.make_async_copy(tail.at[pl.ds(cb, C)], tis[s], isems[s]).wait()
        pltpu.make_async_copy(head.at[pl.ds(cb, C)], his[s], isems[s]).wait()
        pltpu.make_async_copy(ff_hbm.at[pl.ds(cb, C)], ffs[s],
                              isems[s]).wait()

    def p3_compute(s):
        tb, hb, fb = tis[s], his[s], ffs[s]

        def inner(j, c2):
            ob = j * _i32(16 * U)
            for k in range(U):
                o = ob + _i32(16 * k)
                t = tb[pl.ds(o, 16)]
                h = hb[pl.ds(o, 16)]
                ffv = fb[pl.ds(o, 16)]
                plsc.addupdate_scatter(big, [t], ffv)
                plsc.addupdate_scatter(big, [h], -ffv)
            return c2

        lax.fori_loop(_i32(0), _i32(C // (16 * U)), inner, _i32(0))

    run_phase(p3_start_in, p3_wait_in, p3_compute, None, None)

    pltpu.sync_copy(big, accs.at[pl.ds(wid * _i32(NP), NP)])
    plsc.subcore_barrier()
    for r in range(4):
        rb = sid * _i32(RSL) + _i32(r * RSL4)
        pltpu.sync_copy(accs.at[pl.ds(cid * _i32(16 * NP) + rb, RSL4)],
                        vf0.at[pl.ds(0, RSL4)])

        def red_one(k, carry):
            off = (cid * _i32(16) + k) * _i32(NP) + rb
            pltpu.sync_copy(accs.at[pl.ds(off, RSL4)], wf0.at[pl.ds(0, RSL4)])

            def add16(j, c2):
                ob = j * _i32(112)
                for k2 in range(7):
                    o = ob + _i32(16 * k2)
                    vf0[pl.ds(o, 16)] = vf0[pl.ds(o, 16)] + wf0[pl.ds(o, 16)]
                return c2

            lax.fori_loop(_i32(0), _i32(RSL4 // 112), add16, _i32(0))
            return carry

        lax.fori_loop(_i32(1), _i32(16), red_one, _i32(0))
        pltpu.sync_copy(vf0.at[pl.ds(0, RSL4)],
                        partial.at[pl.ds(cid * _i32(NP) + rb, RSL4)])


@jax.jit
def _sc_part(tail, head, vel, flen, ctrl, field):
    mesh = plsc.VectorSubcoreMesh(core_axis_name="c", subcore_axis_name="s")
    f = pl.kernel(
        _sc_body,
        out_type=[
            jax.ShapeDtypeStruct((2 * NP,), jnp.float32),
        ],
        mesh=mesh,
        compiler_params=pltpu.CompilerParams(needs_layout_passes=False),
        scratch_types=[
            pltpu.HBM((N_LINKS,), jnp.int32),
            pltpu.HBM((N_LINKS,), jnp.float32),
            pltpu.HBM((NW * NP,), jnp.float32),
            pltpu.VMEM((NP,), jnp.float32),
            pltpu.VMEM((C,), jnp.int32),
            pltpu.VMEM((C,), jnp.int32),
            pltpu.VMEM((C,), jnp.int32),
            pltpu.VMEM((C,), jnp.int32),
            pltpu.VMEM((C,), jnp.int32),
            pltpu.VMEM((C,), jnp.int32),
            pltpu.VMEM((C,), jnp.float32),
            pltpu.VMEM((C,), jnp.float32),
            pltpu.VMEM((C,), jnp.float32),
            pltpu.VMEM((C,), jnp.float32),
            pltpu.VMEM((C,), jnp.float32),
            pltpu.VMEM((C,), jnp.float32),
            pltpu.SemaphoreType.DMA,
            pltpu.SemaphoreType.DMA,
            pltpu.SemaphoreType.DMA,
            pltpu.SemaphoreType.DMA,
        ],
    )
    return f(tail, head, vel, flen, ctrl, field)


def _tc_combine_body(f_ref, a_ref, p0_ref, p1_ref, dt_ref, o_ref):
    dt = dt_ref[0]
    o_ref[...] = f_ref[...] - dt * (p0_ref[...] + p1_ref[...]) / a_ref[...]


@jax.jit
def _tc_combine(fp, ap, p0, p1, dt):
    return pl.pallas_call(
        _tc_combine_body,
        out_shape=jax.ShapeDtypeStruct((NP // 1024, 1024), jnp.float32),
        in_specs=[
            pl.BlockSpec(memory_space=pltpu.MemorySpace.VMEM),
            pl.BlockSpec(memory_space=pltpu.MemorySpace.VMEM),
            pl.BlockSpec(memory_space=pltpu.MemorySpace.VMEM),
            pl.BlockSpec(memory_space=pltpu.MemorySpace.VMEM),
            pl.BlockSpec(memory_space=pltpu.MemorySpace.SMEM),
        ],
        out_specs=pl.BlockSpec(memory_space=pltpu.MemorySpace.VMEM),
    )(fp, ap, p0, p1, dt)


def kernel(field, control, velocity, edge_index, length_of_face,
           cell_area_at_node, dt):
    tail = edge_index[0].astype(jnp.int32)
    head = edge_index[1].astype(jnp.int32)
    (partial,) = _sc_part(tail, head, velocity, length_of_face,
                          control, field)
    pad = NP - N_NODES
    fp = jnp.reshape(jnp.pad(field, (0, pad)), (NP // 1024, 1024))
    ap = jnp.reshape(jnp.pad(cell_area_at_node, (0, pad),
                             constant_values=jnp.float32(1.0)),
                     (NP // 1024, 1024))
    p0 = jnp.reshape(partial[:NP], (NP // 1024, 1024))
    p1 = jnp.reshape(partial[NP:], (NP // 1024, 1024))
    dt_arr = jnp.reshape(dt.astype(jnp.float32), (1,))
    out = _tc_combine(fp, ap, p0, p1, dt_arr)
    return jnp.reshape(out, (NP,))[:N_NODES]
